# baseline (speedup 1.0000x reference)
import functools

import jax
import jax.numpy as jnp
from jax import lax
from jax.experimental import pallas as pl
from jax.experimental.pallas import tpu as pltpu
from jax.experimental.pallas import tpu_sc as plsc

N = 50000
E = 800000
B = 64
F = 80
R = 40
C = 1024

NC = 2
NS = 16
NW = NC * NS

CHUNK = 128
NCHUNK = E // CHUNK
GROUP = 8
GROUPS = 48

RH = R // NC
RHP = 32
ACC_ROWS = 50176
SLAB = ACC_ROWS // NS
TRASH = ACC_ROWS - 1

BN1 = 2048
GSTEPS1 = 25
BN = 4000
BP = BN // 4
GSTEPS = 13
NP = 12512
NPV = N // 4
AP = ACC_ROWS // 4


def _sc_segment_sum(e3, y0, y1, zrows):
  mesh = plsc.VectorSubcoreMesh(core_axis_name="c", subcore_axis_name="s")

  @functools.partial(
      pl.kernel,
      mesh=mesh,
      compiler_params=pltpu.CompilerParams(use_tc_tiling_on_sc=False),
      out_type=jax.ShapeDtypeStruct((NC, ACC_ROWS, RHP), jnp.bfloat16),
      scratch_types=[
          pltpu.VMEM((GROUP, CHUNK), jnp.int32),
          pltpu.VMEM((GROUP, CHUNK), jnp.int32),
          pltpu.VMEM((GROUP, CHUNK, RHP), jnp.bfloat16),
          pltpu.VMEM((GROUP, CHUNK), jnp.int32),
          pltpu.VMEM((GROUP, CHUNK), jnp.int32),
          pltpu.VMEM((GROUP, CHUNK, RHP), jnp.bfloat16),
          pltpu.VMEM_SHARED((ACC_ROWS, RHP), jnp.bfloat16),
          pltpu.SemaphoreType.DMA,
          pltpu.SemaphoreType.DMA,
          pltpu.SemaphoreType.DMA,
          pltpu.SemaphoreType.DMA,
      ],
  )
  def k(e_hbm, y0_hbm, y1_hbm, z_hbm, out_hbm,
        sidxA, didxA, rowsA, sidxB, didxB, rowsB, acc,
        semA, semB, semSA, semSB):
    c = lax.axis_index("c")
    s = lax.axis_index("s")

    pltpu.sync_copy(z_hbm, acc.at[pl.ds(s * SLAB, SLAB)])
    plsc.subcore_barrier()

    base = 390 * s + jnp.minimum(s, 10)
    tail = jnp.where(s < 10, 391 - GROUPS * GROUP, 390 - GROUPS * GROUP)

    def run(y_hbm):
      def load_fire(g, sidx, didx, rows, sem):
        r0 = base + g * GROUP
        pltpu.sync_copy(e_hbm.at[0, pl.ds(r0, GROUP)], sidx)
        pltpu.sync_copy(e_hbm.at[1, pl.ds(r0, GROUP)], didx)
        for j in range(GROUP):
          pltpu.async_copy(y_hbm.at[sidx.at[j]], rows.at[j], sem)

      def drain_gathers(sidx, rows, sem):
        for j in range(GROUP):
          pltpu.make_async_copy(y_hbm.at[sidx.at[j]], rows.at[j], sem).wait()

      def fire_scatters(didx, rows, sem):
        for j in range(GROUP):
          pltpu.async_copy(rows.at[j], acc.at[didx.at[j]], sem, add=True)

      def drain_scatters(didx, rows, sem):
        for j in range(GROUP):
          pltpu.make_async_copy(rows.at[j], acc.at[didx.at[j]], sem).wait()

      load_fire(0, sidxA, didxA, rowsA, semA)

      def body(i, carry):
        g = i * 2

        @pl.when(g > 0)
        def _():
          drain_scatters(didxB, rowsB, semSB)

        load_fire(g + 1, sidxB, didxB, rowsB, semB)
        drain_gathers(sidxA, rowsA, semA)
        fire_scatters(didxA, rowsA, semSA)
        drain_scatters(didxA, rowsA, semSA)

        @pl.when(g + 2 < GROUPS)
        def _():
          load_fire(g + 2, sidxA, didxA, rowsA, semA)

        drain_gathers(sidxB, rowsB, semB)
        fire_scatters(didxB, rowsB, semSB)
        return carry

      lax.fori_loop(0, GROUPS // 2, body, 0)
      drain_scatters(didxB, rowsB, semSB)

      def tail_body(t, carry):
        r = base + GROUPS * GROUP + t
        pltpu.sync_copy(e_hbm.at[0, r], sidxA.at[0])
        pltpu.sync_copy(e_hbm.at[1, r], didxA.at[0])
        pltpu.async_copy(y_hbm.at[sidxA.at[0]], rowsA.at[0], semA).wait()
        pltpu.sync_copy(rowsA.at[0], acc.at[didxA.at[0]], add=True)
        return carry

      lax.fori_loop(0, tail, tail_body, 0)

    @pl.when(c == 0)
    def _():
      run(y0_hbm)

    @pl.when(c == 1)
    def _():
      run(y1_hbm)

    plsc.subcore_barrier()

    pltpu.sync_copy(acc.at[pl.ds(s * SLAB, SLAB)],
                    out_hbm.at[c, pl.ds(s * SLAB, SLAB)])

  return k(e3, y0, y1, zrows)


def _mm_rule(x4_ref, w0_ref, w1_ref, o0_ref, o1_ref):
  x4 = x4_ref[...]
  o0_ref[...] = jnp.dot(
      x4, w0_ref[...], preferred_element_type=jnp.float32).astype(jnp.bfloat16)
  o1_ref[...] = jnp.dot(
      x4, w1_ref[...], preferred_element_type=jnp.float32).astype(jnp.bfloat16)


def _fused_body(y0_ref, y1_ref, a0_ref, a1_ref, gid_ref, br0_ref, br1_ref,
                wc4_ref, bc4_ref,
                w1_ref, b1_ref, w2_ref, b2_ref, w3_ref, b3_ref,
                w6_ref, b6_ref, w7_ref, b7_ref, ph_ref, o_ref, pooled):
  i = pl.program_id(0)
  h0 = jnp.maximum(
      y0_ref[...].astype(jnp.float32) + a0_ref[0].astype(jnp.float32)
      + br0_ref[...], 0.0)
  h1 = jnp.maximum(
      y1_ref[...].astype(jnp.float32) + a1_ref[0].astype(jnp.float32)
      + br1_ref[...], 0.0)
  validr = (i * BP + lax.broadcasted_iota(jnp.int32, (BP, 1), 0)) < NPV
  hc = jnp.where(validr, jnp.concatenate([h0, h1], axis=1), 0.0)
  gq = jnp.dot(hc.astype(jnp.bfloat16), wc4_ref[...],
               preferred_element_type=jnp.float32)
  gq = jnp.maximum(gq + bc4_ref[...], 0.0).astype(jnp.bfloat16)
  validc = (i * BP + lax.broadcasted_iota(jnp.int32, (B, BP), 1)) < NPV
  biota = lax.broadcasted_iota(jnp.int32, (B, BP), 0)
  part = jnp.zeros((B, C), jnp.float32)
  for a in range(4):
    oh = jnp.where((gid_ref[0, a, :][None, :] == biota) & validc,
                   1.0, 0.0).astype(jnp.bfloat16)
    part += jnp.dot(oh, gq[:, C * a:C * (a + 1)],
                    preferred_element_type=jnp.float32)

  @pl.when(i == 0)
  def _():
    pooled[...] = jnp.zeros_like(pooled)

  pooled[...] += part

  @pl.when(i == pl.num_programs(0) - 1)
  def _():
    p = pooled[...]
    d1 = jax.nn.sigmoid(
        jnp.dot(p, w1_ref[...], preferred_element_type=jnp.float32)
        + b1_ref[...])
    d2 = jax.nn.sigmoid(
        jnp.dot(d1, w2_ref[...], preferred_element_type=jnp.float32)
        + b2_ref[...])
    z3 = (jnp.dot(d2, w3_ref[...], preferred_element_type=jnp.float32)
          + b3_ref[...])
    d3 = jax.nn.softmax(z3, axis=-1)
    mv = jnp.maximum(
        jnp.dot(d3, w6_ref[...], preferred_element_type=jnp.float32)
        + b6_ref[...], 0.0)
    merged = (mv * w7_ref[0:1, :]
              + jnp.dot(ph_ref[...], w7_ref[1:16, :],
                        preferred_element_type=jnp.float32)
              + b7_ref[...])
    o_ref[...] = jnp.maximum(merged, 0.0)


def kernel(x, edge_index, graph_ids, physics_info, W_rule, b_rule,
           W_conv, b_conv, W1, b1, W2, b2, W3, b3, W6, b6, W7, b7):
  x4 = x.reshape(NPV, 4 * F)
  eye4 = jnp.eye(4, dtype=jnp.float32)
  W40 = jnp.kron(eye4, jnp.pad(W_rule[:, :RH], ((0, 0), (0, RHP - RH))))
  W41 = jnp.kron(eye4, jnp.pad(W_rule[:, RH:], ((0, 0), (0, RHP - RH))))
  BP1 = BN1 // 4
  y0p, y1p = pl.pallas_call(
      _mm_rule,
      grid=(GSTEPS1,),
      in_specs=[
          pl.BlockSpec((BP1, 4 * F), lambda i: (i, 0)),
          pl.BlockSpec((4 * F, 128), lambda i: (0, 0)),
          pl.BlockSpec((4 * F, 128), lambda i: (0, 0)),
      ],
      out_specs=[pl.BlockSpec((BP1, 128), lambda i: (i, 0)),
                 pl.BlockSpec((BP1, 128), lambda i: (i, 0))],
      out_shape=[jax.ShapeDtypeStruct((NP, 128), jnp.bfloat16),
                 jax.ShapeDtypeStruct((NP, 128), jnp.bfloat16)],
  )(x4, W40, W41)
  y0 = y0p.reshape(NP * 4, RHP)
  y1 = y1p.reshape(NP * 4, RHP)

  e3 = edge_index.reshape(2, NCHUNK, CHUNK)
  zrows = jnp.zeros((SLAB, RHP), jnp.bfloat16)
  agg = _sc_segment_sum(e3, y0, y1, zrows)

  aggp = agg.reshape(NC, AP, 128)
  gid4 = jnp.pad(graph_ids, (0, GSTEPS * BN - N)).reshape(
      GSTEPS, BP, 4).transpose(0, 2, 1)
  br0 = jnp.tile(jnp.pad(b_rule[:RH], (0, RHP - RH)), 4).reshape(1, 128)
  br1 = jnp.tile(jnp.pad(b_rule[RH:], (0, RHP - RH)), 4).reshape(1, 128)
  wc4 = jnp.concatenate(
      [jnp.kron(eye4, jnp.pad(W_conv[:RH], ((0, RHP - RH), (0, 0)))),
       jnp.kron(eye4, jnp.pad(W_conv[RH:], ((0, RHP - RH), (0, 0))))],
      axis=0).astype(jnp.bfloat16)
  bc4 = jnp.tile(b_conv, 4).reshape(1, 4 * C)
  out = pl.pallas_call(
      _fused_body,
      grid=(GSTEPS,),
      in_specs=[
          pl.BlockSpec((BP, 128), lambda i: (i, 0)),
          pl.BlockSpec((BP, 128), lambda i: (i, 0)),
          pl.BlockSpec((1, BP, 128), lambda i: (0, i, 0)),
          pl.BlockSpec((1, BP, 128), lambda i: (1, i, 0)),
          pl.BlockSpec((1, 4, BP), lambda i: (i, 0, 0)),
          pl.BlockSpec((1, 128), lambda i: (0, 0)),
          pl.BlockSpec((1, 128), lambda i: (0, 0)),
          pl.BlockSpec((2 * 4 * RHP, 4 * C), lambda i: (0, 0)),
          pl.BlockSpec((1, 4 * C), lambda i: (0, 0)),
          pl.BlockSpec((C, 300), lambda i: (0, 0)),
          pl.BlockSpec((1, 300), lambda i: (0, 0)),
          pl.BlockSpec((300, 100), lambda i: (0, 0)),
          pl.BlockSpec((1, 100), lambda i: (0, 0)),
          pl.BlockSpec((100, 20), lambda i: (0, 0)),
          pl.BlockSpec((1, 20), lambda i: (0, 0)),
          pl.BlockSpec((20, 1), lambda i: (0, 0)),
          pl.BlockSpec((1, 1), lambda i: (0, 0)),
          pl.BlockSpec((16, 1), lambda i: (0, 0)),
          pl.BlockSpec((1, 1), lambda i: (0, 0)),
          pl.BlockSpec((B, 15), lambda i: (0, 0)),
      ],
      out_specs=pl.BlockSpec((B, 1), lambda i: (0, 0)),
      out_shape=jax.ShapeDtypeStruct((B, 1), jnp.float32),
      scratch_shapes=[pltpu.VMEM((B, C), jnp.float32)],
  )(y0p, y1p, aggp, aggp, gid4, br0, br1, wc4, bc4,
    W1, b1.reshape(1, 300), W2, b2.reshape(1, 100),
    W3, b3.reshape(1, 20), W6, b6.reshape(1, 1), W7, b7.reshape(1, 1),
    physics_info)
  return out


# speedup vs baseline: 1.0261x; 1.0261x over previous
import functools

import jax
import jax.numpy as jnp
from jax import lax
from jax.experimental import pallas as pl
from jax.experimental.pallas import tpu as pltpu
from jax.experimental.pallas import tpu_sc as plsc

N = 50000
E = 800000
B = 64
F = 80
R = 40
C = 1024

NC = 2
NS = 16
NW = NC * NS

CHUNK = 125
GROUP = 8
ROWS_PER_TILE = E // NS // CHUNK
GROUPS = ROWS_PER_TILE // GROUP

RH = R // NC
RHP = 32
ACC_ROWS = 50176
SLAB = ACC_ROWS // NS
TRASH = ACC_ROWS - 1

BN1 = 2048
GSTEPS1 = 25
BN = 4000
BP = BN // 4
GSTEPS = 13
NP = 12512
NPV = N // 4
AP = ACC_ROWS // 4


def _sc_segment_sum(src2d, dst2d, y0, y1, zrows):
  mesh = plsc.VectorSubcoreMesh(core_axis_name="c", subcore_axis_name="s")

  @functools.partial(
      pl.kernel,
      mesh=mesh,
      compiler_params=pltpu.CompilerParams(use_tc_tiling_on_sc=False),
      out_type=jax.ShapeDtypeStruct((NC, ACC_ROWS, RHP), jnp.bfloat16),
      scratch_types=[
          pltpu.VMEM((GROUP, CHUNK), jnp.int32),
          pltpu.VMEM((GROUP, CHUNK), jnp.int32),
          pltpu.VMEM((GROUP, CHUNK, RHP), jnp.bfloat16),
          pltpu.VMEM((GROUP, CHUNK), jnp.int32),
          pltpu.VMEM((GROUP, CHUNK), jnp.int32),
          pltpu.VMEM((GROUP, CHUNK, RHP), jnp.bfloat16),
          pltpu.VMEM_SHARED((ACC_ROWS, RHP), jnp.bfloat16),
          pltpu.SemaphoreType.DMA,
          pltpu.SemaphoreType.DMA,
          pltpu.SemaphoreType.DMA,
          pltpu.SemaphoreType.DMA,
      ],
  )
  def k(src_hbm, dst_hbm, y0_hbm, y1_hbm, z_hbm, out_hbm,
        sidxA, didxA, rowsA, sidxB, didxB, rowsB, acc,
        semA, semB, semSA, semSB):
    c = lax.axis_index("c")
    s = lax.axis_index("s")

    pltpu.sync_copy(z_hbm, acc.at[pl.ds(s * SLAB, SLAB)])
    plsc.subcore_barrier()

    base = s * ROWS_PER_TILE

    def run(y_hbm):
      def load_fire(g, sidx, didx, rows, sem):
        r0 = base + g * GROUP
        pltpu.sync_copy(src_hbm.at[pl.ds(r0, GROUP)], sidx)
        pltpu.sync_copy(dst_hbm.at[pl.ds(r0, GROUP)], didx)
        for j in range(GROUP):
          pltpu.async_copy(y_hbm.at[sidx.at[j]], rows.at[j], sem)

      def drain_gathers(sidx, rows, sem):
        for j in range(GROUP):
          pltpu.make_async_copy(y_hbm.at[sidx.at[j]], rows.at[j], sem).wait()

      def fire_scatters(didx, rows, sem):
        for j in range(GROUP):
          pltpu.async_copy(rows.at[j], acc.at[didx.at[j]], sem, add=True)

      def drain_scatters(didx, rows, sem):
        for j in range(GROUP):
          pltpu.make_async_copy(rows.at[j], acc.at[didx.at[j]], sem).wait()

      load_fire(0, sidxA, didxA, rowsA, semA)

      def body(i, carry):
        g = i * 2

        @pl.when(g > 0)
        def _():
          drain_scatters(didxB, rowsB, semSB)

        load_fire(g + 1, sidxB, didxB, rowsB, semB)
        drain_gathers(sidxA, rowsA, semA)
        fire_scatters(didxA, rowsA, semSA)
        drain_scatters(didxA, rowsA, semSA)

        @pl.when(g + 2 < GROUPS)
        def _():
          load_fire(g + 2, sidxA, didxA, rowsA, semA)

        drain_gathers(sidxB, rowsB, semB)
        fire_scatters(didxB, rowsB, semSB)
        return carry

      lax.fori_loop(0, GROUPS // 2, body, 0)
      drain_scatters(didxB, rowsB, semSB)

    @pl.when(c == 0)
    def _():
      run(y0_hbm)

    @pl.when(c == 1)
    def _():
      run(y1_hbm)

    plsc.subcore_barrier()

    pltpu.sync_copy(acc.at[pl.ds(s * SLAB, SLAB)],
                    out_hbm.at[c, pl.ds(s * SLAB, SLAB)])

  return k(src2d, dst2d, y0, y1, zrows)


def _mm_rule(x4_ref, w0_ref, w1_ref, o0_ref, o1_ref):
  x4 = x4_ref[...]
  o0_ref[...] = jnp.dot(
      x4, w0_ref[...], preferred_element_type=jnp.float32).astype(jnp.bfloat16)
  o1_ref[...] = jnp.dot(
      x4, w1_ref[...], preferred_element_type=jnp.float32).astype(jnp.bfloat16)


def _fused_body(y0_ref, y1_ref, a0_ref, a1_ref, gid_ref, br0_ref, br1_ref,
                wc4_ref, bc4_ref,
                w1_ref, b1_ref, w2_ref, b2_ref, w3_ref, b3_ref,
                w6_ref, b6_ref, w7_ref, b7_ref, ph_ref, o_ref, pooled):
  i = pl.program_id(0)
  h0 = jnp.maximum(
      y0_ref[...].astype(jnp.float32) + a0_ref[0].astype(jnp.float32)
      + br0_ref[...], 0.0)
  h1 = jnp.maximum(
      y1_ref[...].astype(jnp.float32) + a1_ref[0].astype(jnp.float32)
      + br1_ref[...], 0.0)
  validr = (i * BP + lax.broadcasted_iota(jnp.int32, (BP, 1), 0)) < NPV
  hc = jnp.where(validr, jnp.concatenate([h0, h1], axis=1), 0.0)
  gq = jnp.dot(hc.astype(jnp.bfloat16), wc4_ref[...],
               preferred_element_type=jnp.float32)
  gq = jnp.maximum(gq + bc4_ref[...], 0.0).astype(jnp.bfloat16)
  validc = (i * BP + lax.broadcasted_iota(jnp.int32, (B, BP), 1)) < NPV
  biota = lax.broadcasted_iota(jnp.int32, (B, BP), 0)
  part = jnp.zeros((B, C), jnp.float32)
  for a in range(4):
    oh = jnp.where((gid_ref[0, a, :][None, :] == biota) & validc,
                   1.0, 0.0).astype(jnp.bfloat16)
    part += jnp.dot(oh, gq[:, C * a:C * (a + 1)],
                    preferred_element_type=jnp.float32)

  @pl.when(i == 0)
  def _():
    pooled[...] = jnp.zeros_like(pooled)

  pooled[...] += part

  @pl.when(i == pl.num_programs(0) - 1)
  def _():
    p = pooled[...]
    d1 = jax.nn.sigmoid(
        jnp.dot(p, w1_ref[...], preferred_element_type=jnp.float32)
        + b1_ref[...])
    d2 = jax.nn.sigmoid(
        jnp.dot(d1, w2_ref[...], preferred_element_type=jnp.float32)
        + b2_ref[...])
    z3 = (jnp.dot(d2, w3_ref[...], preferred_element_type=jnp.float32)
          + b3_ref[...])
    d3 = jax.nn.softmax(z3, axis=-1)
    mv = jnp.maximum(
        jnp.dot(d3, w6_ref[...], preferred_element_type=jnp.float32)
        + b6_ref[...], 0.0)
    merged = (mv * w7_ref[0:1, :]
              + jnp.dot(ph_ref[...], w7_ref[1:16, :],
                        preferred_element_type=jnp.float32)
              + b7_ref[...])
    o_ref[...] = jnp.maximum(merged, 0.0)


def kernel(x, edge_index, graph_ids, physics_info, W_rule, b_rule,
           W_conv, b_conv, W1, b1, W2, b2, W3, b3, W6, b6, W7, b7):
  x4 = x.reshape(NPV, 4 * F)
  eye4 = jnp.eye(4, dtype=jnp.float32)
  W40 = jnp.kron(eye4, jnp.pad(W_rule[:, :RH], ((0, 0), (0, RHP - RH))))
  W41 = jnp.kron(eye4, jnp.pad(W_rule[:, RH:], ((0, 0), (0, RHP - RH))))
  BP1 = BN1 // 4
  y0p, y1p = pl.pallas_call(
      _mm_rule,
      grid=(GSTEPS1,),
      in_specs=[
          pl.BlockSpec((BP1, 4 * F), lambda i: (i, 0)),
          pl.BlockSpec((4 * F, 128), lambda i: (0, 0)),
          pl.BlockSpec((4 * F, 128), lambda i: (0, 0)),
      ],
      out_specs=[pl.BlockSpec((BP1, 128), lambda i: (i, 0)),
                 pl.BlockSpec((BP1, 128), lambda i: (i, 0))],
      out_shape=[jax.ShapeDtypeStruct((NP, 128), jnp.bfloat16),
                 jax.ShapeDtypeStruct((NP, 128), jnp.bfloat16)],
  )(x4, W40, W41)
  y0 = y0p.reshape(NP * 4, RHP)
  y1 = y1p.reshape(NP * 4, RHP)

  src_p = edge_index[0].reshape(-1, CHUNK)
  dst_p = edge_index[1].reshape(-1, CHUNK)
  zrows = jnp.zeros((SLAB, RHP), jnp.bfloat16)
  agg = _sc_segment_sum(src_p, dst_p, y0, y1, zrows)

  aggp = agg.reshape(NC, AP, 128)
  gid4 = jnp.pad(graph_ids, (0, GSTEPS * BN - N)).reshape(
      GSTEPS, BP, 4).transpose(0, 2, 1)
  br0 = jnp.tile(jnp.pad(b_rule[:RH], (0, RHP - RH)), 4).reshape(1, 128)
  br1 = jnp.tile(jnp.pad(b_rule[RH:], (0, RHP - RH)), 4).reshape(1, 128)
  wc4 = jnp.concatenate(
      [jnp.kron(eye4, jnp.pad(W_conv[:RH], ((0, RHP - RH), (0, 0)))),
       jnp.kron(eye4, jnp.pad(W_conv[RH:], ((0, RHP - RH), (0, 0))))],
      axis=0).astype(jnp.bfloat16)
  bc4 = jnp.tile(b_conv, 4).reshape(1, 4 * C)
  out = pl.pallas_call(
      _fused_body,
      grid=(GSTEPS,),
      in_specs=[
          pl.BlockSpec((BP, 128), lambda i: (i, 0)),
          pl.BlockSpec((BP, 128), lambda i: (i, 0)),
          pl.BlockSpec((1, BP, 128), lambda i: (0, i, 0)),
          pl.BlockSpec((1, BP, 128), lambda i: (1, i, 0)),
          pl.BlockSpec((1, 4, BP), lambda i: (i, 0, 0)),
          pl.BlockSpec((1, 128), lambda i: (0, 0)),
          pl.BlockSpec((1, 128), lambda i: (0, 0)),
          pl.BlockSpec((2 * 4 * RHP, 4 * C), lambda i: (0, 0)),
          pl.BlockSpec((1, 4 * C), lambda i: (0, 0)),
          pl.BlockSpec((C, 300), lambda i: (0, 0)),
          pl.BlockSpec((1, 300), lambda i: (0, 0)),
          pl.BlockSpec((300, 100), lambda i: (0, 0)),
          pl.BlockSpec((1, 100), lambda i: (0, 0)),
          pl.BlockSpec((100, 20), lambda i: (0, 0)),
          pl.BlockSpec((1, 20), lambda i: (0, 0)),
          pl.BlockSpec((20, 1), lambda i: (0, 0)),
          pl.BlockSpec((1, 1), lambda i: (0, 0)),
          pl.BlockSpec((16, 1), lambda i: (0, 0)),
          pl.BlockSpec((1, 1), lambda i: (0, 0)),
          pl.BlockSpec((B, 15), lambda i: (0, 0)),
      ],
      out_specs=pl.BlockSpec((B, 1), lambda i: (0, 0)),
      out_shape=jax.ShapeDtypeStruct((B, 1), jnp.float32),
      scratch_shapes=[pltpu.VMEM((B, C), jnp.float32)],
  )(y0p, y1p, aggp, aggp, gid4, br0, br1, wc4, bc4,
    W1, b1.reshape(1, 300), W2, b2.reshape(1, 100),
    W3, b3.reshape(1, 20), W6, b6.reshape(1, 1), W7, b7.reshape(1, 1),
    physics_info)
  return out
